# Initial kernel scaffold; baseline (speedup 1.0000x reference)
import jax
import jax.numpy as jnp
from jax.experimental import pallas as pl


def kernel(x, weight, ln_weight, ln_bias):
    raise NotImplementedError("write your pallas kernel here")


# speedup vs baseline: 1.1330x; 1.1330x over previous
import functools

import jax
import jax.numpy as jnp
import numpy as np
from jax import lax
from jax.experimental import pallas as pl
from jax.experimental.pallas import tpu as pltpu
from jax.experimental.pallas import tpu_sc as plsc

D = 64
L16 = 16
NV = D // L16
EPS = 1e-5

_info = plsc.get_sparse_core_info()
NC, NS = _info.num_cores, _info.num_subcores
NW = NC * NS

CHUNK = 256
IDXW = 128
GPC = CHUNK // IDXW
STAGE = 1024
CPS = STAGE // CHUNK


def _rsqrt_nr(x):
    i = lax.bitcast_convert_type(x, jnp.int32)
    i = jnp.int32(0x5F3759DF) - (i >> 1)
    y = lax.bitcast_convert_type(i, jnp.float32)
    for _ in range(3):
        y = y * (1.5 - 0.5 * x * y * y)
    return y


_GDN = lax.GatherDimensionNumbers(
    offset_dims=(), collapsed_slice_dims=(0,), start_index_map=(0,))


def _lane_allsum(v, perms):
    for p in perms:
        pv = lax.gather(v, p, _GDN, slice_sizes=(1,),
                        mode=lax.GatherScatterMode.PROMISE_IN_BOUNDS)
        v = v + pv
    return v


def _make_kernel(n_rows):
    assert n_rows % (NW * STAGE) == 0
    rows_per_w = n_rows // NW
    n_groups = rows_per_w // STAGE
    mesh = plsc.VectorSubcoreMesh(core_axis_name="c", subcore_axis_name="s")

    @functools.partial(
        pl.kernel,
        mesh=mesh,
        compiler_params=pltpu.CompilerParams(use_tc_tiling_on_sc=False),
        out_type=jax.ShapeDtypeStruct((n_rows, D), jnp.float32),
        scratch_types=[
            pltpu.VMEM((STAGE // IDXW, IDXW), jnp.int32),
            pltpu.VMEM((CHUNK, D), jnp.float32),
            pltpu.VMEM((D,), jnp.float32),
            pltpu.VMEM((D,), jnp.float32),
            pltpu.SemaphoreType.DMA,
        ],
    )
    def emb_ln(x_hbm, w_hbm, lnw_hbm, lnb_hbm, out_hbm,
               idx_v, rows_v, lnw_v, lnb_v, sem):
        wid = lax.axis_index("s") * NC + lax.axis_index("c")
        base = wid * rows_per_w

        pltpu.sync_copy(lnw_hbm, lnw_v)
        pltpu.sync_copy(lnb_hbm, lnb_v)
        w_vecs = [lnw_v[pl.ds(k * L16, L16)] for k in range(NV)]
        b_vecs = [lnb_v[pl.ds(k * L16, L16)] for k in range(NV)]
        lane = lax.iota(jnp.int32, L16)
        perms = [(lane ^ (1 << b))[:, None] for b in range(4)]

        def group_body(g, _):
            grow0 = base + g * STAGE
            goff = pl.multiple_of(grow0 // IDXW, 8)
            pltpu.sync_copy(x_hbm.at[pl.ds(goff, STAGE // IDXW)], idx_v)
            for c in range(CPS):
                row0 = grow0 + c * CHUNK
                for j in range(GPC):
                    pltpu.async_copy(
                        w_hbm.at[idx_v.at[c * GPC + j]],
                        rows_v.at[pl.ds(j * IDXW, IDXW)], sem).wait()

                def row_body(r, _):
                    vs = [rows_v[r, pl.ds(k * L16, L16)] for k in range(NV)]
                    s = vs[0] + vs[1] + vs[2] + vs[3]
                    q = (vs[0] * vs[0] + vs[1] * vs[1]
                         + vs[2] * vs[2] + vs[3] * vs[3])
                    mean = _lane_allsum(s, perms) * (1.0 / D)
                    ex2 = _lane_allsum(q, perms) * (1.0 / D)
                    rstd = _rsqrt_nr(ex2 - mean * mean + EPS)
                    for k in range(NV):
                        sl = pl.ds(k * L16, L16)
                        rows_v[r, sl] = ((vs[k] - mean) * rstd * w_vecs[k]
                                         + b_vecs[k])
                    return 0

                lax.fori_loop(0, CHUNK, row_body, 0)

                pltpu.sync_copy(rows_v, out_hbm.at[pl.ds(row0, CHUNK)])
            return 0

        lax.fori_loop(0, n_groups, group_body, 0)

    return emb_ln


def kernel(x, weight, ln_weight, ln_bias):
    b, h = x.shape
    n = b * h
    x2 = x.reshape(n // IDXW, IDXW).astype(jnp.int32)
    out = _make_kernel(n)(x2, weight, ln_weight, ln_bias)
    return out.reshape(b, h, D)


# speedup vs baseline: 1.5611x; 1.3779x over previous
import functools

import jax
import jax.numpy as jnp
import numpy as np
from jax import lax
from jax.experimental import pallas as pl
from jax.experimental.pallas import tpu as pltpu
from jax.experimental.pallas import tpu_sc as plsc

D = 64
L16 = 16
NV = D // L16
EPS = 1e-5

_info = plsc.get_sparse_core_info()
NC, NS = _info.num_cores, _info.num_subcores
NW = NC * NS

CHUNK = 256
IDXW = 128
GPC = CHUNK // IDXW
STAGE = 1024
CPS = STAGE // CHUNK


def _rsqrt_nr(x):
    i = lax.bitcast_convert_type(x, jnp.int32)
    i = jnp.int32(0x5F3759DF) - (i >> 1)
    y = lax.bitcast_convert_type(i, jnp.float32)
    for _ in range(3):
        y = y * (1.5 - 0.5 * x * y * y)
    return y


_GDN = lax.GatherDimensionNumbers(
    offset_dims=(), collapsed_slice_dims=(0,), start_index_map=(0,))


def _lane_allsum(v, perms):
    for p in perms:
        pv = lax.gather(v, p, _GDN, slice_sizes=(1,),
                        mode=lax.GatherScatterMode.PROMISE_IN_BOUNDS)
        v = v + pv
    return v


def _make_kernel(n_rows):
    assert n_rows % (NW * STAGE) == 0
    rows_per_w = n_rows // NW
    n_groups = rows_per_w // STAGE
    mesh = plsc.VectorSubcoreMesh(core_axis_name="c", subcore_axis_name="s")

    @functools.partial(
        pl.kernel,
        mesh=mesh,
        compiler_params=pltpu.CompilerParams(use_tc_tiling_on_sc=False),
        out_type=jax.ShapeDtypeStruct((n_rows, D), jnp.float32),
        scratch_types=[
            pltpu.VMEM((STAGE // IDXW, IDXW), jnp.int32),
            pltpu.VMEM((CHUNK, D), jnp.float32),
            pltpu.VMEM((D,), jnp.float32),
            pltpu.VMEM((D,), jnp.float32),
            pltpu.SemaphoreType.DMA,
        ],
    )
    def emb_ln(x_hbm, w_hbm, lnw_hbm, lnb_hbm, out_hbm,
               idx_v, rows_v, lnw_v, lnb_v, sem):
        wid = lax.axis_index("s") * NC + lax.axis_index("c")
        base = wid * rows_per_w

        pltpu.sync_copy(lnw_hbm, lnw_v)
        pltpu.sync_copy(lnb_hbm, lnb_v)
        w_vecs = [lnw_v[pl.ds(k * L16, L16)] for k in range(NV)]
        b_vecs = [lnb_v[pl.ds(k * L16, L16)] for k in range(NV)]
        lane = lax.iota(jnp.int32, L16)
        perms = [(lane ^ (1 << b))[:, None] for b in range(4)]

        def group_body(g, _):
            grow0 = base + g * STAGE
            goff = pl.multiple_of(grow0 // IDXW, 8)
            pltpu.sync_copy(x_hbm.at[pl.ds(goff, STAGE // IDXW)], idx_v)
            for c in range(CPS):
                row0 = grow0 + c * CHUNK
                for j in range(GPC):
                    pltpu.async_copy(
                        w_hbm.at[idx_v.at[c * GPC + j]],
                        rows_v.at[pl.ds(j * IDXW, IDXW)], sem).wait()

                @plsc.parallel_loop(0, CHUNK, unroll=8)
                def row_body(r):
                    vs = [rows_v[r, pl.ds(k * L16, L16)] for k in range(NV)]
                    s = vs[0] + vs[1] + vs[2] + vs[3]
                    q = (vs[0] * vs[0] + vs[1] * vs[1]
                         + vs[2] * vs[2] + vs[3] * vs[3])
                    mean = _lane_allsum(s, perms) * (1.0 / D)
                    ex2 = _lane_allsum(q, perms) * (1.0 / D)
                    rstd = _rsqrt_nr(ex2 - mean * mean + EPS)
                    for k in range(NV):
                        sl = pl.ds(k * L16, L16)
                        rows_v[r, sl] = ((vs[k] - mean) * rstd * w_vecs[k]
                                         + b_vecs[k])

                pltpu.sync_copy(rows_v, out_hbm.at[pl.ds(row0, CHUNK)])
            return 0

        lax.fori_loop(0, n_groups, group_body, 0)

    return emb_ln


def kernel(x, weight, ln_weight, ln_bias):
    b, h = x.shape
    n = b * h
    x2 = x.reshape(n // IDXW, IDXW).astype(jnp.int32)
    out = _make_kernel(n)(x2, weight, ln_weight, ln_bias)
    return out.reshape(b, h, D)
